# baseline (speedup 1.0000x reference)
import functools

import jax
import jax.numpy as jnp
from jax import lax
from jax.experimental import pallas as pl
from jax.experimental.pallas import tpu as pltpu
from jax.experimental.pallas import tpu_sc as plsc

N_NODES = 30000
N_EDGES = 480000
CHUNK = 128
N_CHUNKS = N_EDGES // CHUNK
N_PAD = 30208
ROWS_PER_TILE = N_PAD // 16

_MESH = plsc.VectorSubcoreMesh(core_axis_name="c", subcore_axis_name="s")


DDEPTH = 4


def _deg_kernel(dst_hbm, ones_hbm, zeros_hbm, out_hbm, idx_v, ones_v, acc_sh,
                *sems):
    esems, ssems = sems[:DDEPTH], sems[DDEPTH:]
    c = lax.axis_index("c")
    s = lax.axis_index("s")
    w = s * 2 + c
    pltpu.sync_copy(zeros_hbm.at[pl.ds(s * ROWS_PER_TILE, ROWS_PER_TILE)],
                    acc_sh.at[pl.ds(s * ROWS_PER_TILE, ROWS_PER_TILE)])
    pltpu.sync_copy(ones_hbm, ones_v)

    def guard(t, fn):
        @pl.when(jnp.logical_and(t >= 0, w + 32 * t < N_CHUNKS))
        def _():
            fn()

    def base(t):
        return (w + 32 * t) * CHUNK

    def fire_stage(t, b):
        guard(t, lambda: pltpu.async_copy(
            dst_hbm.at[pl.ds(base(t), CHUNK)], idx_v.at[b], esems[b]))

    def wait_stage(t, b):
        guard(t, lambda: pltpu.make_async_copy(
            dst_hbm.at[pl.ds(base(t), CHUNK)], idx_v.at[b], esems[b]).wait())

    def fire_scatter(t, b):
        guard(t, lambda: pltpu.async_copy(
            ones_v, acc_sh.at[idx_v.at[b]], ssems[b], add=True))

    def wait_scatter(t, b):
        guard(t, lambda: pltpu.make_async_copy(
            ones_v, acc_sh.at[idx_v.at[b]], ssems[b]).wait())

    plsc.subcore_barrier()
    for t in range(DDEPTH - 1):
        fire_stage(t, t)

    def body(jj, _):
        for u in range(DDEPTH):
            j = DDEPTH * jj + u
            b = u
            b3 = (u + 3) % DDEPTH
            wait_stage(j, b)
            fire_scatter(j, b)
            wait_scatter(j - 1, b3)
            fire_stage(j + 3, b3)
        return 0

    nit = (N_CHUNKS + 31) // 32
    lax.fori_loop(0, (nit + 1 + DDEPTH - 1) // DDEPTH, body, 0)
    plsc.subcore_barrier()
    pltpu.sync_copy(acc_sh.at[pl.ds(s * ROWS_PER_TILE, ROWS_PER_TILE)],
                    out_hbm.at[c, pl.ds(s * ROWS_PER_TILE, ROWS_PER_TILE)])


def _make_deg_call():
    return functools.partial(
        pl.kernel,
        mesh=_MESH,
        out_type=jax.ShapeDtypeStruct((2, N_PAD, 2), jnp.float32),
        scratch_types=[
            pltpu.VMEM((DDEPTH, CHUNK), jnp.int32),
            pltpu.VMEM((CHUNK, 2), jnp.float32),
            pltpu.VMEM_SHARED((N_PAD, 2), jnp.float32),
        ] + [pltpu.SemaphoreType.DMA] * (2 * DDEPTH),
        compiler_params=pltpu.CompilerParams(use_tc_tiling_on_sc=False),
    )(_deg_kernel)


NIT = (N_CHUNKS + 15) // 16
DEPTH = 4


def _scatter_kernel(width, qoff, src_hbm, dst_hbm, y_hbm, zeros_hbm, out_hbm,
                    sidx_v, sdst_v, gidx_v, rows_v, acc_sh, *sems):
    esems, gsems, ssems = sems[0:DEPTH], sems[DEPTH:2 * DEPTH], sems[2 * DEPTH:]
    c = lax.axis_index("c")
    s = lax.axis_index("s")
    pltpu.sync_copy(zeros_hbm.at[pl.ds(s * ROWS_PER_TILE, ROWS_PER_TILE)],
                    acc_sh.at[pl.ds(s * ROWS_PER_TILE, ROWS_PER_TILE)])
    off = (qoff + c) * N_NODES

    def guard(t, fn):
        @pl.when(jnp.logical_and(t >= 0, s + 16 * t < N_CHUNKS))
        def _():
            fn()

    def base(t):
        return (s + 16 * t) * CHUNK

    def fire_stage(t, b):
        guard(t, lambda: (
            pltpu.async_copy(src_hbm.at[pl.ds(base(t), CHUNK)], sidx_v.at[b],
                             esems[b]),
            pltpu.async_copy(dst_hbm.at[pl.ds(base(t), CHUNK)], sdst_v.at[b],
                             esems[b])))

    def wait_stage(t, b):
        guard(t, lambda: (
            pltpu.make_async_copy(src_hbm.at[pl.ds(base(t), CHUNK)],
                                  sidx_v.at[b], esems[b]).wait(),
            pltpu.make_async_copy(dst_hbm.at[pl.ds(base(t), CHUNK)],
                                  sdst_v.at[b], esems[b]).wait()))

    def compute_gidx(t, b):
        def go():
            for i in range(CHUNK // 16):
                gidx_v[b, pl.ds(i * 16, 16)] = sidx_v[b, pl.ds(i * 16, 16)] + off
        guard(t, go)

    def fire_gather(t, b):
        guard(t, lambda: pltpu.async_copy(
            y_hbm.at[gidx_v.at[b]], rows_v.at[b], gsems[b]))

    def wait_gather(t, b):
        guard(t, lambda: pltpu.make_async_copy(
            y_hbm.at[gidx_v.at[b]], rows_v.at[b], gsems[b]).wait())

    def fire_scatter(t, b):
        guard(t, lambda: pltpu.async_copy(
            rows_v.at[b], acc_sh.at[sdst_v.at[b]], ssems[b], add=True))

    def wait_scatter(t, b):
        guard(t, lambda: pltpu.make_async_copy(
            rows_v.at[b], acc_sh.at[sdst_v.at[b]], ssems[b]).wait())

    plsc.subcore_barrier()
    for t in range(DEPTH - 1):
        fire_stage(t, t)
    for t in range(DEPTH - 2):
        wait_stage(t, t)
        compute_gidx(t, t)
        fire_gather(t, t)

    def body(jj, _):
        for u in range(DEPTH):
            j = DEPTH * jj + u
            b = u
            b2 = (u + 2) % DEPTH
            b3 = (u + 3) % DEPTH
            wait_gather(j, b)
            fire_scatter(j, b)
            wait_stage(j + 2, b2)
            compute_gidx(j + 2, b2)
            fire_gather(j + 2, b2)
            wait_scatter(j - 1, b3)
            fire_stage(j + 3, b3)
        return 0

    lax.fori_loop(0, (NIT + 1 + DEPTH - 1) // DEPTH, body, 0)
    plsc.subcore_barrier()
    pltpu.sync_copy(acc_sh.at[pl.ds(s * ROWS_PER_TILE, ROWS_PER_TILE)],
                    out_hbm.at[c, pl.ds(s * ROWS_PER_TILE, ROWS_PER_TILE)])


def _make_scatter_call(width, qoff, nq):
    return functools.partial(
        pl.kernel,
        mesh=_MESH,
        out_type=jax.ShapeDtypeStruct((nq, N_PAD, width), jnp.float32),
        scratch_types=[
            pltpu.VMEM((DEPTH, CHUNK), jnp.int32),
            pltpu.VMEM((DEPTH, CHUNK), jnp.int32),
            pltpu.VMEM((DEPTH, CHUNK), jnp.int32),
            pltpu.VMEM((DEPTH, CHUNK, width), jnp.float32),
            pltpu.VMEM_SHARED((N_PAD, width), jnp.float32),
        ] + [pltpu.SemaphoreType.DMA] * (3 * DEPTH),
        compiler_params=pltpu.CompilerParams(use_tc_tiling_on_sc=False),
    )(functools.partial(_scatter_kernel, width, qoff))


_RB = 600


def _dinv_from(degp):
    cnt = degp[0, :, 0] + degp[1, :, 0]
    return lax.rsqrt(1.0 + cnt)


def _y1_body(x_ref, w_ref, degp_ref, out_ref, dinv_ref):
    dinv = _dinv_from(degp_ref[...])
    dinv_ref[...] = jnp.broadcast_to(dinv[:, None], dinv_ref.shape)
    y = jnp.dot(x_ref[...], w_ref[...], preferred_element_type=jnp.float32)
    y = y * dinv[:, None]
    for q in range(4):
        out_ref[q] = y[:, 32 * q:32 * q + 32]


def _y2a_body(sa_ref, y1_ref, dinv_ref, w_ref, b_ref, out_ref):
    dinv = dinv_ref[:, 0][:, None]
    hq = [jnp.maximum((sa_ref[i] + y1_ref[i]) * dinv
                      + b_ref[32 * i:32 * i + 32], 0.0)
          for i in range(2)]
    h1a = jnp.concatenate(hq, axis=1)
    y2p = jnp.dot(h1a, w_ref[...], preferred_element_type=jnp.float32)
    out_ref[0] = y2p[:, :32]
    out_ref[1] = y2p[:, 32:]


def _y2b_body(sb_ref, y1_ref, dinv_ref, w_ref, b_ref, y2p_ref, out_ref):
    dinv = dinv_ref[:, 0][:, None]
    hq = [jnp.maximum((sb_ref[i] + y1_ref[i]) * dinv
                      + b_ref[64 + 32 * i:64 + 32 * i + 32], 0.0)
          for i in range(2)]
    h1b = jnp.concatenate(hq, axis=1)
    y2p = jnp.concatenate([y2p_ref[0], y2p_ref[1]], axis=1)
    y2 = (y2p + jnp.dot(h1b, w_ref[...],
                        preferred_element_type=jnp.float32)) * dinv
    out_ref[0] = y2[:, :32]
    out_ref[1] = y2[:, 32:]


def _h2_body(s2_ref, y2_ref, dinv_ref, b_ref, out_ref):
    dinv = dinv_ref[:, 0][:, None]
    ha = jnp.maximum((s2_ref[0] + y2_ref[0]) * dinv + b_ref[:32], 0.0)
    hb = jnp.maximum((s2_ref[1] + y2_ref[1]) * dinv + b_ref[32:], 0.0)
    out_ref[...] = jnp.concatenate([ha, hb], axis=1)


def _fc_body(h_ref, w_ref, b_ref, out_ref):
    out_ref[...] = (jnp.dot(h_ref[...], w_ref[...],
                            preferred_element_type=jnp.float32) + b_ref[...])


def kernel(x, edge_index, W1, b1, W2, b2, Wfc, bfc):
    src = edge_index[0].astype(jnp.int32)
    dst = edge_index[1].astype(jnp.int32)

    ones16 = jnp.ones((CHUNK, 2), jnp.float32)
    zeros16 = jnp.zeros((N_PAD, 2), jnp.float32)
    zeros32 = jnp.zeros((N_PAD, 32), jnp.float32)

    degp = _make_deg_call()(dst, ones16, zeros16)

    nblk = N_NODES // _RB
    y1, dinv = pl.pallas_call(
        _y1_body,
        grid=(nblk,),
        in_specs=[
            pl.BlockSpec((_RB, 128), lambda r: (r, 0)),
            pl.BlockSpec((128, 128), lambda r: (0, 0)),
            pl.BlockSpec((2, _RB, 2), lambda r: (0, r, 0)),
        ],
        out_specs=[
            pl.BlockSpec((4, _RB, 32), lambda r: (0, r, 0)),
            pl.BlockSpec((_RB, 8), lambda r: (r, 0)),
        ],
        out_shape=[
            jax.ShapeDtypeStruct((4, N_NODES, 32), jnp.float32),
            jax.ShapeDtypeStruct((N_NODES, 8), jnp.float32),
        ],
    )(x, W1, degp)

    y1f = y1.reshape(4 * N_NODES, 32)
    sa = _make_scatter_call(32, 0, 2)(src, dst, y1f, zeros32)
    sb = _make_scatter_call(32, 2, 2)(src, dst, y1f, zeros32)

    y2p = pl.pallas_call(
        _y2a_body,
        grid=(nblk,),
        in_specs=[
            pl.BlockSpec((2, _RB, 32), lambda r: (0, r, 0)),
            pl.BlockSpec((2, _RB, 32), lambda r: (0, r, 0)),
            pl.BlockSpec((_RB, 8), lambda r: (r, 0)),
            pl.BlockSpec((64, 64), lambda r: (0, 0)),
            pl.BlockSpec((128,), lambda r: (0,)),
        ],
        out_specs=pl.BlockSpec((2, _RB, 32), lambda r: (0, r, 0)),
        out_shape=jax.ShapeDtypeStruct((2, N_NODES, 32), jnp.float32),
    )(sa, y1, dinv, W2, b1)

    y2 = pl.pallas_call(
        _y2b_body,
        grid=(nblk,),
        in_specs=[
            pl.BlockSpec((2, _RB, 32), lambda r: (0, r, 0)),
            pl.BlockSpec((2, _RB, 32), lambda r: (1, r, 0)),
            pl.BlockSpec((_RB, 8), lambda r: (r, 0)),
            pl.BlockSpec((64, 64), lambda r: (1, 0)),
            pl.BlockSpec((128,), lambda r: (0,)),
            pl.BlockSpec((2, _RB, 32), lambda r: (0, r, 0)),
        ],
        out_specs=pl.BlockSpec((2, _RB, 32), lambda r: (0, r, 0)),
        out_shape=jax.ShapeDtypeStruct((2, N_NODES, 32), jnp.float32),
    )(sb, y1, dinv, W2, b1, y2p)

    s2 = _make_scatter_call(32, 0, 2)(src, dst, y2.reshape(2 * N_NODES, 32),
                                      zeros32)

    h2 = pl.pallas_call(
        _h2_body,
        grid=(nblk,),
        in_specs=[
            pl.BlockSpec((2, _RB, 32), lambda r: (0, r, 0)),
            pl.BlockSpec((2, _RB, 32), lambda r: (0, r, 0)),
            pl.BlockSpec((_RB, 8), lambda r: (r, 0)),
            pl.BlockSpec((64,), lambda r: (0,)),
        ],
        out_specs=pl.BlockSpec((_RB, 64), lambda r: (r, 0)),
        out_shape=jax.ShapeDtypeStruct((N_NODES, 64), jnp.float32),
    )(s2, y2, dinv, b2)

    h2r = h2.reshape(1000, 1920)
    out = pl.pallas_call(
        _fc_body,
        grid=(5,),
        in_specs=[
            pl.BlockSpec((200, 1920), lambda j: (j, 0)),
            pl.BlockSpec((1920, 1728), lambda j: (0, 0)),
            pl.BlockSpec((1728,), lambda j: (0,)),
        ],
        out_specs=pl.BlockSpec((200, 1728), lambda j: (j, 0)),
        out_shape=jax.ShapeDtypeStruct((1000, 1728), jnp.float32),
    )(h2r, Wfc, bfc)
    return out


# speedup vs baseline: 1.3483x; 1.3483x over previous
import functools

import jax
import jax.numpy as jnp
from jax import lax
from jax.experimental import pallas as pl
from jax.experimental.pallas import tpu as pltpu
from jax.experimental.pallas import tpu_sc as plsc

N_NODES = 30000
N_EDGES = 480000
CHUNK = 128
N_CHUNKS = N_EDGES // CHUNK
N_PAD = 30208
ROWS_PER_TILE = N_PAD // 16
ZROWS = 118

_MESH = plsc.VectorSubcoreMesh(core_axis_name="c", subcore_axis_name="s")


DDEPTH = 4


def _deg_kernel(dst_hbm, ones_hbm, zeros_hbm, out_hbm, idx_v, ones_v, acc_sh,
                *sems):
    esems, ssems = sems[:DDEPTH], sems[DDEPTH:]
    c = lax.axis_index("c")
    s = lax.axis_index("s")
    w = s * 2 + c
    pltpu.sync_copy(zeros_hbm.at[pl.ds(s * ROWS_PER_TILE, ROWS_PER_TILE)],
                    acc_sh.at[pl.ds(s * ROWS_PER_TILE, ROWS_PER_TILE)])
    pltpu.sync_copy(ones_hbm, ones_v)

    def guard(t, fn):
        @pl.when(jnp.logical_and(t >= 0, w + 32 * t < N_CHUNKS))
        def _():
            fn()

    def base(t):
        return (w + 32 * t) * CHUNK

    def fire_stage(t, b):
        guard(t, lambda: pltpu.async_copy(
            dst_hbm.at[pl.ds(base(t), CHUNK)], idx_v.at[b], esems[b]))

    def wait_stage(t, b):
        guard(t, lambda: pltpu.make_async_copy(
            dst_hbm.at[pl.ds(base(t), CHUNK)], idx_v.at[b], esems[b]).wait())

    def fire_scatter(t, b):
        guard(t, lambda: pltpu.async_copy(
            ones_v, acc_sh.at[idx_v.at[b]], ssems[b], add=True))

    def wait_scatter(t, b):
        guard(t, lambda: pltpu.make_async_copy(
            ones_v, acc_sh.at[idx_v.at[b]], ssems[b]).wait())

    plsc.subcore_barrier()
    for t in range(DDEPTH - 1):
        fire_stage(t, t)

    def body(jj, _):
        for u in range(DDEPTH):
            j = DDEPTH * jj + u
            b = u
            b3 = (u + 3) % DDEPTH
            wait_stage(j, b)
            fire_scatter(j, b)
            wait_scatter(j - 1, b3)
            fire_stage(j + 3, b3)
        return 0

    nit = (N_CHUNKS + 31) // 32
    lax.fori_loop(0, (nit + 1 + DDEPTH - 1) // DDEPTH, body, 0)
    plsc.subcore_barrier()
    pltpu.sync_copy(acc_sh.at[pl.ds(s * ROWS_PER_TILE, ROWS_PER_TILE)],
                    out_hbm.at[pl.ds(s * ROWS_PER_TILE, ROWS_PER_TILE),
                               pl.ds(8 * c, 8)])


def _make_deg_call():
    return functools.partial(
        pl.kernel,
        mesh=_MESH,
        out_type=jax.ShapeDtypeStruct((N_PAD, 128), jnp.float32),
        scratch_types=[
            pltpu.VMEM((DDEPTH, CHUNK), jnp.int32),
            pltpu.VMEM((CHUNK, 8), jnp.float32),
            pltpu.VMEM_SHARED((N_PAD, 8), jnp.float32),
        ] + [pltpu.SemaphoreType.DMA] * (2 * DDEPTH),
        compiler_params=pltpu.CompilerParams(use_tc_tiling_on_sc=False),
    )(_deg_kernel)


NIT = (N_CHUNKS + 15) // 16
DEPTH = 4


def _scatter_kernel(nphase, src_hbm, dst_hbm, y_hbm, zeros_hbm, out_hbm,
                    sidx_v, sdst_v, gidx_v, rows_v, zv, acc_sh, *sems):
    esems, gsems, ssems = sems[0:DEPTH], sems[DEPTH:2 * DEPTH], sems[2 * DEPTH:]
    c = lax.axis_index("c")
    s = lax.axis_index("s")
    pltpu.sync_copy(zeros_hbm, zv)

    def zero_acc():
        for k in range(ROWS_PER_TILE // ZROWS):
            pltpu.sync_copy(
                zv, acc_sh.at[pl.ds(s * ROWS_PER_TILE + k * ZROWS, ZROWS)])

    def guard(t, fn):
        @pl.when(jnp.logical_and(t >= 0, s + 16 * t < N_CHUNKS))
        def _():
            fn()

    def base(t):
        return (s + 16 * t) * CHUNK

    def fire_stage(t, b):
        guard(t, lambda: (
            pltpu.async_copy(src_hbm.at[pl.ds(base(t), CHUNK)], sidx_v.at[b],
                             esems[b]),
            pltpu.async_copy(dst_hbm.at[pl.ds(base(t), CHUNK)], sdst_v.at[b],
                             esems[b])))

    def wait_stage(t, b):
        guard(t, lambda: (
            pltpu.make_async_copy(src_hbm.at[pl.ds(base(t), CHUNK)],
                                  sidx_v.at[b], esems[b]).wait(),
            pltpu.make_async_copy(dst_hbm.at[pl.ds(base(t), CHUNK)],
                                  sdst_v.at[b], esems[b]).wait()))

    def make_phase(p):
        q = 2 * p + c

        def compute_gidx(t, b):
            def go():
                for i in range(CHUNK // 16):
                    gidx_v[b, pl.ds(i * 16, 16)] = (
                        sidx_v[b, pl.ds(i * 16, 16)] * 4 + q)
            guard(t, go)

        def fire_gather(t, b):
            guard(t, lambda: pltpu.async_copy(
                y_hbm.at[gidx_v.at[b]], rows_v.at[b], gsems[b]))

        def wait_gather(t, b):
            guard(t, lambda: pltpu.make_async_copy(
                y_hbm.at[gidx_v.at[b]], rows_v.at[b], gsems[b]).wait())

        def fire_scatter(t, b):
            guard(t, lambda: pltpu.async_copy(
                rows_v.at[b], acc_sh.at[sdst_v.at[b]], ssems[b], add=True))

        def wait_scatter(t, b):
            guard(t, lambda: pltpu.make_async_copy(
                rows_v.at[b], acc_sh.at[sdst_v.at[b]], ssems[b]).wait())

        for t in range(DEPTH - 1):
            fire_stage(t, t)
        for t in range(DEPTH - 2):
            wait_stage(t, t)
            compute_gidx(t, t)
            fire_gather(t, t)

        def body(jj, _):
            for u in range(DEPTH):
                j = DEPTH * jj + u
                b = u
                b2 = (u + 2) % DEPTH
                b3 = (u + 3) % DEPTH
                wait_gather(j, b)
                fire_scatter(j, b)
                wait_stage(j + 2, b2)
                compute_gidx(j + 2, b2)
                fire_gather(j + 2, b2)
                wait_scatter(j - 1, b3)
                fire_stage(j + 3, b3)
            return 0

        lax.fori_loop(0, (NIT + 1 + DEPTH - 1) // DEPTH, body, 0)
        plsc.subcore_barrier()
        pltpu.sync_copy(acc_sh.at[pl.ds(s * ROWS_PER_TILE, ROWS_PER_TILE)],
                        out_hbm.at[pl.ds(s * ROWS_PER_TILE, ROWS_PER_TILE),
                                   pl.ds(32 * q, 32)])

    zero_acc()
    plsc.subcore_barrier()
    for p in range(nphase):
        if p:
            zero_acc()
            plsc.subcore_barrier()
        make_phase(p)


def _make_scatter_call(nphase):
    return functools.partial(
        pl.kernel,
        mesh=_MESH,
        out_type=jax.ShapeDtypeStruct((N_PAD, 128), jnp.float32),
        scratch_types=[
            pltpu.VMEM((DEPTH, CHUNK), jnp.int32),
            pltpu.VMEM((DEPTH, CHUNK), jnp.int32),
            pltpu.VMEM((DEPTH, CHUNK), jnp.int32),
            pltpu.VMEM((DEPTH, CHUNK, 32), jnp.float32),
            pltpu.VMEM((ZROWS, 32), jnp.float32),
            pltpu.VMEM_SHARED((N_PAD, 32), jnp.float32),
        ] + [pltpu.SemaphoreType.DMA] * (3 * DEPTH),
        compiler_params=pltpu.CompilerParams(use_tc_tiling_on_sc=False),
    )(functools.partial(_scatter_kernel, nphase))


_RB = 600


def _y1_body(x_ref, w_ref, deg_ref, out_ref, dinv_ref):
    cnt = deg_ref[:, 0] + deg_ref[:, 8]
    dinv = lax.rsqrt(1.0 + cnt)
    dinv_ref[...] = jnp.broadcast_to(dinv[:, None], dinv_ref.shape)
    y = jnp.dot(x_ref[...], w_ref[...], preferred_element_type=jnp.float32)
    out_ref[...] = y * dinv[:, None]


def _y2_body(s1_ref, y1_ref, dinv_ref, w_ref, b_ref, out_ref):
    dinv = dinv_ref[:, 0][:, None]
    h1 = jnp.maximum((s1_ref[...] + y1_ref[...]) * dinv + b_ref[...], 0.0)
    y2 = jnp.dot(h1, w_ref[...], preferred_element_type=jnp.float32) * dinv
    out_ref[...] = jnp.concatenate(
        [y2, jnp.zeros((y2.shape[0], 64), jnp.float32)], axis=1)


def _h2_body(s2_ref, y2_ref, dinv_ref, b_ref, out_ref):
    dinv = dinv_ref[:, 0][:, None]
    out_ref[...] = jnp.maximum(
        (s2_ref[:, :64] + y2_ref[:, :64]) * dinv + b_ref[...], 0.0)


def _fc_body(h_ref, w_ref, b_ref, out_ref):
    out_ref[...] = (jnp.dot(h_ref[...], w_ref[...],
                            preferred_element_type=jnp.float32) + b_ref[...])


def kernel(x, edge_index, W1, b1, W2, b2, Wfc, bfc):
    src = edge_index[0].astype(jnp.int32)
    dst = edge_index[1].astype(jnp.int32)

    ones2 = jnp.ones((CHUNK, 8), jnp.float32)
    zeros2 = jnp.zeros((N_PAD, 8), jnp.float32)
    zeros32 = jnp.zeros((ZROWS, 32), jnp.float32)

    degp = _make_deg_call()(dst, ones2, zeros2)

    nblk = N_NODES // _RB
    y1, dinv = pl.pallas_call(
        _y1_body,
        grid=(nblk,),
        in_specs=[
            pl.BlockSpec((_RB, 128), lambda r: (r, 0)),
            pl.BlockSpec((128, 128), lambda r: (0, 0)),
            pl.BlockSpec((_RB, 128), lambda r: (r, 0)),
        ],
        out_specs=[
            pl.BlockSpec((_RB, 128), lambda r: (r, 0)),
            pl.BlockSpec((_RB, 8), lambda r: (r, 0)),
        ],
        out_shape=[
            jax.ShapeDtypeStruct((N_NODES, 128), jnp.float32),
            jax.ShapeDtypeStruct((N_NODES, 8), jnp.float32),
        ],
    )(x, W1, degp)

    s1 = _make_scatter_call(2)(src, dst, y1.reshape(4 * N_NODES, 32), zeros32)

    y2 = pl.pallas_call(
        _y2_body,
        grid=(nblk,),
        in_specs=[
            pl.BlockSpec((_RB, 128), lambda r: (r, 0)),
            pl.BlockSpec((_RB, 128), lambda r: (r, 0)),
            pl.BlockSpec((_RB, 8), lambda r: (r, 0)),
            pl.BlockSpec((128, 64), lambda r: (0, 0)),
            pl.BlockSpec((128,), lambda r: (0,)),
        ],
        out_specs=pl.BlockSpec((_RB, 128), lambda r: (r, 0)),
        out_shape=jax.ShapeDtypeStruct((N_NODES, 128), jnp.float32),
    )(s1, y1, dinv, W2, b1)

    s2 = _make_scatter_call(1)(src, dst, y2.reshape(4 * N_NODES, 32), zeros32)

    h2 = pl.pallas_call(
        _h2_body,
        grid=(nblk,),
        in_specs=[
            pl.BlockSpec((_RB, 128), lambda r: (r, 0)),
            pl.BlockSpec((_RB, 128), lambda r: (r, 0)),
            pl.BlockSpec((_RB, 8), lambda r: (r, 0)),
            pl.BlockSpec((64,), lambda r: (0,)),
        ],
        out_specs=pl.BlockSpec((_RB, 64), lambda r: (r, 0)),
        out_shape=jax.ShapeDtypeStruct((N_NODES, 64), jnp.float32),
    )(s2, y2, dinv, b2)

    h2r = h2.reshape(1000, 1920)
    out = pl.pallas_call(
        _fc_body,
        grid=(5,),
        in_specs=[
            pl.BlockSpec((200, 1920), lambda j: (j, 0)),
            pl.BlockSpec((1920, 1728), lambda j: (0, 0)),
            pl.BlockSpec((1728,), lambda j: (0,)),
        ],
        out_specs=pl.BlockSpec((200, 1728), lambda j: (j, 0)),
        out_shape=jax.ShapeDtypeStruct((1000, 1728), jnp.float32),
    )(h2r, Wfc, bfc)
    return out


# speedup vs baseline: 1.3744x; 1.0193x over previous
import functools

import jax
import jax.numpy as jnp
from jax import lax
from jax.experimental import pallas as pl
from jax.experimental.pallas import tpu as pltpu
from jax.experimental.pallas import tpu_sc as plsc

N_NODES = 30000
N_EDGES = 480000
CHUNK = 128
N_CHUNKS = N_EDGES // CHUNK
N_PAD = 30208
ROWS_PER_TILE = N_PAD // 16
ZROWS = 118

_MESH = plsc.VectorSubcoreMesh(core_axis_name="c", subcore_axis_name="s")


DDEPTH = 4


def _deg_kernel(edge_hbm, ones_hbm, zeros_hbm, out_hbm, idx_v, ones_v, acc_sh,
                *sems):
    esems, ssems = sems[:DDEPTH], sems[DDEPTH:]
    c = lax.axis_index("c")
    s = lax.axis_index("s")
    w = s * 2 + c
    pltpu.sync_copy(zeros_hbm.at[pl.ds(s * ROWS_PER_TILE, ROWS_PER_TILE)],
                    acc_sh.at[pl.ds(s * ROWS_PER_TILE, ROWS_PER_TILE)])
    pltpu.sync_copy(ones_hbm, ones_v)

    def guard(t, fn):
        @pl.when(jnp.logical_and(t >= 0, w + 32 * t < N_CHUNKS))
        def _():
            fn()

    def base(t):
        return (w + 32 * t) * CHUNK

    def fire_stage(t, b):
        guard(t, lambda: pltpu.async_copy(
            edge_hbm.at[1, pl.ds(base(t), CHUNK)], idx_v.at[b], esems[b]))

    def wait_stage(t, b):
        guard(t, lambda: pltpu.make_async_copy(
            edge_hbm.at[1, pl.ds(base(t), CHUNK)], idx_v.at[b],
            esems[b]).wait())

    def fire_scatter(t, b):
        guard(t, lambda: pltpu.async_copy(
            ones_v, acc_sh.at[idx_v.at[b]], ssems[b], add=True))

    def wait_scatter(t, b):
        guard(t, lambda: pltpu.make_async_copy(
            ones_v, acc_sh.at[idx_v.at[b]], ssems[b]).wait())

    plsc.subcore_barrier()
    for t in range(DDEPTH - 1):
        fire_stage(t, t)

    def body(jj, _):
        for u in range(DDEPTH):
            j = DDEPTH * jj + u
            b = u
            b3 = (u + 3) % DDEPTH
            wait_stage(j, b)
            fire_scatter(j, b)
            wait_scatter(j - 1, b3)
            fire_stage(j + 3, b3)
        return 0

    nit = (N_CHUNKS + 31) // 32
    lax.fori_loop(0, (nit + 1 + DDEPTH - 1) // DDEPTH, body, 0)
    plsc.subcore_barrier()
    pltpu.sync_copy(acc_sh.at[pl.ds(s * ROWS_PER_TILE, ROWS_PER_TILE)],
                    out_hbm.at[pl.ds(s * ROWS_PER_TILE, ROWS_PER_TILE),
                               pl.ds(8 * c, 8)])


def _make_deg_call():
    return functools.partial(
        pl.kernel,
        mesh=_MESH,
        out_type=jax.ShapeDtypeStruct((N_PAD, 128), jnp.float32),
        scratch_types=[
            pltpu.VMEM((DDEPTH, CHUNK), jnp.int32),
            pltpu.VMEM((CHUNK, 8), jnp.float32),
            pltpu.VMEM_SHARED((N_PAD, 8), jnp.float32),
        ] + [pltpu.SemaphoreType.DMA] * (2 * DDEPTH),
        compiler_params=pltpu.CompilerParams(use_tc_tiling_on_sc=False),
    )(_deg_kernel)


NIT = (N_CHUNKS + 15) // 16
DEPTH = 4


def _scatter_kernel(nphase, edge_hbm, y_hbm, zeros_hbm, out_hbm,
                    sidx_v, sdst_v, gidx_v, rows_v, zv, acc_sh, *sems):
    esems, gsems, ssems = sems[0:DEPTH], sems[DEPTH:2 * DEPTH], sems[2 * DEPTH:]
    c = lax.axis_index("c")
    s = lax.axis_index("s")
    pltpu.sync_copy(zeros_hbm, zv)

    def zero_acc():
        for k in range(ROWS_PER_TILE // ZROWS):
            pltpu.sync_copy(
                zv, acc_sh.at[pl.ds(s * ROWS_PER_TILE + k * ZROWS, ZROWS)])

    def guard(t, fn):
        @pl.when(jnp.logical_and(t >= 0, s + 16 * t < N_CHUNKS))
        def _():
            fn()

    def base(t):
        return (s + 16 * t) * CHUNK

    def fire_stage(t, b):
        guard(t, lambda: (
            pltpu.async_copy(edge_hbm.at[0, pl.ds(base(t), CHUNK)],
                             sidx_v.at[b], esems[b]),
            pltpu.async_copy(edge_hbm.at[1, pl.ds(base(t), CHUNK)],
                             sdst_v.at[b], esems[b])))

    def wait_stage(t, b):
        guard(t, lambda: (
            pltpu.make_async_copy(edge_hbm.at[0, pl.ds(base(t), CHUNK)],
                                  sidx_v.at[b], esems[b]).wait(),
            pltpu.make_async_copy(edge_hbm.at[1, pl.ds(base(t), CHUNK)],
                                  sdst_v.at[b], esems[b]).wait()))

    def make_phase(p):
        q = 2 * p + c

        def compute_gidx(t, b):
            def go():
                for i in range(CHUNK // 16):
                    gidx_v[b, pl.ds(i * 16, 16)] = (
                        sidx_v[b, pl.ds(i * 16, 16)] * 4 + q)
            guard(t, go)

        def fire_gather(t, b):
            guard(t, lambda: pltpu.async_copy(
                y_hbm.at[gidx_v.at[b]], rows_v.at[b], gsems[b]))

        def wait_gather(t, b):
            guard(t, lambda: pltpu.make_async_copy(
                y_hbm.at[gidx_v.at[b]], rows_v.at[b], gsems[b]).wait())

        def fire_scatter(t, b):
            guard(t, lambda: pltpu.async_copy(
                rows_v.at[b], acc_sh.at[sdst_v.at[b]], ssems[b], add=True))

        def wait_scatter(t, b):
            guard(t, lambda: pltpu.make_async_copy(
                rows_v.at[b], acc_sh.at[sdst_v.at[b]], ssems[b]).wait())

        for t in range(DEPTH - 1):
            fire_stage(t, t)
        for t in range(DEPTH - 2):
            wait_stage(t, t)
            compute_gidx(t, t)
            fire_gather(t, t)

        def body(jj, _):
            for u in range(DEPTH):
                j = DEPTH * jj + u
                b = u
                b2 = (u + 2) % DEPTH
                b3 = (u + 3) % DEPTH
                wait_gather(j, b)
                fire_scatter(j, b)
                wait_stage(j + 2, b2)
                compute_gidx(j + 2, b2)
                fire_gather(j + 2, b2)
                wait_scatter(j - 1, b3)
                fire_stage(j + 3, b3)
            return 0

        lax.fori_loop(0, (NIT + 1 + DEPTH - 1) // DEPTH, body, 0)
        plsc.subcore_barrier()
        pltpu.sync_copy(acc_sh.at[pl.ds(s * ROWS_PER_TILE, ROWS_PER_TILE)],
                        out_hbm.at[pl.ds(s * ROWS_PER_TILE, ROWS_PER_TILE),
                                   pl.ds(32 * q, 32)])

    zero_acc()
    plsc.subcore_barrier()
    for p in range(nphase):
        if p:
            zero_acc()
            plsc.subcore_barrier()
        make_phase(p)


def _make_scatter_call(nphase):
    return functools.partial(
        pl.kernel,
        mesh=_MESH,
        out_type=jax.ShapeDtypeStruct((N_PAD, 128), jnp.float32),
        scratch_types=[
            pltpu.VMEM((DEPTH, CHUNK), jnp.int32),
            pltpu.VMEM((DEPTH, CHUNK), jnp.int32),
            pltpu.VMEM((DEPTH, CHUNK), jnp.int32),
            pltpu.VMEM((DEPTH, CHUNK, 32), jnp.float32),
            pltpu.VMEM((ZROWS, 32), jnp.float32),
            pltpu.VMEM_SHARED((N_PAD, 32), jnp.float32),
        ] + [pltpu.SemaphoreType.DMA] * (3 * DEPTH),
        compiler_params=pltpu.CompilerParams(use_tc_tiling_on_sc=False),
    )(functools.partial(_scatter_kernel, nphase))


_RB = 600


def _y1_body(x_ref, w_ref, deg_ref, out_ref, dinv_ref):
    cnt = deg_ref[:, 0] + deg_ref[:, 8]
    dinv = lax.rsqrt(1.0 + cnt)
    dinv_ref[...] = jnp.broadcast_to(dinv[:, None], dinv_ref.shape)
    y = jnp.dot(x_ref[...], w_ref[...], preferred_element_type=jnp.float32)
    out_ref[...] = y * dinv[:, None]


def _y2_body(s1_ref, y1_ref, dinv_ref, w_ref, b_ref, out_ref):
    dinv = dinv_ref[:, 0][:, None]
    h1 = jnp.maximum((s1_ref[...] + y1_ref[...]) * dinv + b_ref[...], 0.0)
    y2 = jnp.dot(h1, w_ref[...], preferred_element_type=jnp.float32) * dinv
    out_ref[...] = jnp.concatenate(
        [y2, jnp.zeros((y2.shape[0], 64), jnp.float32)], axis=1)


def _h2_body(s2_ref, y2_ref, dinv_ref, b_ref, out_ref):
    dinv = dinv_ref[:, 0][:, None]
    out_ref[...] = jnp.maximum(
        (s2_ref[:, :64] + y2_ref[:, :64]) * dinv + b_ref[...], 0.0)


def _fc_body(h_ref, w_ref, b_ref, out_ref):
    out_ref[...] = (jnp.dot(h_ref[...], w_ref[...],
                            preferred_element_type=jnp.float32) + b_ref[...])


def kernel(x, edge_index, W1, b1, W2, b2, Wfc, bfc):
    edges = edge_index.astype(jnp.int32)

    ones2 = jnp.ones((CHUNK, 8), jnp.float32)
    zeros2 = jnp.zeros((N_PAD, 8), jnp.float32)
    zeros32 = jnp.zeros((ZROWS, 32), jnp.float32)

    degp = _make_deg_call()(edges, ones2, zeros2)

    nblk = N_NODES // _RB
    y1, dinv = pl.pallas_call(
        _y1_body,
        grid=(nblk,),
        in_specs=[
            pl.BlockSpec((_RB, 128), lambda r: (r, 0)),
            pl.BlockSpec((128, 128), lambda r: (0, 0)),
            pl.BlockSpec((_RB, 128), lambda r: (r, 0)),
        ],
        out_specs=[
            pl.BlockSpec((_RB, 128), lambda r: (r, 0)),
            pl.BlockSpec((_RB, 8), lambda r: (r, 0)),
        ],
        out_shape=[
            jax.ShapeDtypeStruct((N_NODES, 128), jnp.float32),
            jax.ShapeDtypeStruct((N_NODES, 8), jnp.float32),
        ],
    )(x, W1, degp)

    s1 = _make_scatter_call(2)(edges, y1.reshape(4 * N_NODES, 32), zeros32)

    y2 = pl.pallas_call(
        _y2_body,
        grid=(nblk,),
        in_specs=[
            pl.BlockSpec((_RB, 128), lambda r: (r, 0)),
            pl.BlockSpec((_RB, 128), lambda r: (r, 0)),
            pl.BlockSpec((_RB, 8), lambda r: (r, 0)),
            pl.BlockSpec((128, 64), lambda r: (0, 0)),
            pl.BlockSpec((128,), lambda r: (0,)),
        ],
        out_specs=pl.BlockSpec((_RB, 128), lambda r: (r, 0)),
        out_shape=jax.ShapeDtypeStruct((N_NODES, 128), jnp.float32),
    )(s1, y1, dinv, W2, b1)

    s2 = _make_scatter_call(1)(edges, y2.reshape(4 * N_NODES, 32), zeros32)

    h2 = pl.pallas_call(
        _h2_body,
        grid=(nblk,),
        in_specs=[
            pl.BlockSpec((_RB, 128), lambda r: (r, 0)),
            pl.BlockSpec((_RB, 128), lambda r: (r, 0)),
            pl.BlockSpec((_RB, 8), lambda r: (r, 0)),
            pl.BlockSpec((64,), lambda r: (0,)),
        ],
        out_specs=pl.BlockSpec((_RB, 64), lambda r: (r, 0)),
        out_shape=jax.ShapeDtypeStruct((N_NODES, 64), jnp.float32),
    )(s2, y2, dinv, b2)

    h2r = h2.reshape(1000, 1920)
    out = pl.pallas_call(
        _fc_body,
        grid=(5,),
        in_specs=[
            pl.BlockSpec((200, 1920), lambda j: (j, 0)),
            pl.BlockSpec((1920, 1728), lambda j: (0, 0)),
            pl.BlockSpec((1728,), lambda j: (0,)),
        ],
        out_specs=pl.BlockSpec((200, 1728), lambda j: (j, 0)),
        out_shape=jax.ShapeDtypeStruct((1000, 1728), jnp.float32),
    )(h2r, Wfc, bfc)
    return out
